# baseline (speedup 1.0000x reference)
import functools

import jax
import jax.numpy as jnp
from jax import lax
from jax.experimental import pallas as pl
from jax.experimental.pallas import tpu as pltpu
from jax.experimental.pallas import tpu_sc as plsc

B = 16384
H = 50
D = 32
NC = 2
NS = 16
NW = NC * NS
RB = B // NW
TPAD = RB + 8
UNROLL = 8


def _sc_gather(x, data):
    mesh = plsc.VectorSubcoreMesh(core_axis_name="c", subcore_axis_name="s")

    @functools.partial(
        pl.kernel,
        out_type=jax.ShapeDtypeStruct((H, D, B), jnp.float32),
        mesh=mesh,
        scratch_types=[
            pltpu.VMEM((RB, H), jnp.int32),
            pltpu.VMEM((H, RB), jnp.int32),
            pltpu.VMEM((2, RB, D), jnp.float32),
            pltpu.VMEM((D, TPAD), jnp.float32),
            pltpu.SemaphoreType.DMA,
            pltpu.SemaphoreType.DMA,
        ],
        compiler_params=pltpu.CompilerParams(
            use_tc_tiling_on_sc=False, needs_layout_passes=False
        ),
    )
    def k(x_hbm, data_hbm, outT_hbm, idxBH_v, idxT_v, rows_v, trans_v,
          gsem0, gsem1):
        wid = lax.axis_index("s") * NC + lax.axis_index("c")
        b0 = wid * RB
        pltpu.sync_copy(x_hbm.at[pl.ds(b0, RB)], idxBH_v)
        lanes = lax.iota(jnp.int32, 16)

        def t_body(i, c):
            h = i // (RB // 64)
            blk4 = i % (RB // 64)
            for u in range(4):
                bb = (blk4 * 4 + u) * 16
                v = plsc.load_gather(idxBH_v, [lanes + bb, jnp.full((16,), 0, jnp.int32) + h])
                idxT_v[h, pl.ds(bb, 16)] = v
            return c

        lax.fori_loop(0, H * (RB // 64), t_body, 0)

        def fire(h, buf, sem):
            pltpu.async_copy(data_hbm.at[idxT_v.at[h]], rows_v.at[buf], sem)

        def drain(buf, sem):
            pltpu.make_async_copy(
                outT_hbm.at[0, :, pl.ds(b0, RB)], rows_v.at[buf], sem
            ).wait()

        def process(h, buf):
            def b_body(bb, c2):
                b = bb * UNROLL
                for u in range(UNROLL):
                    v0 = rows_v[buf, b + u, pl.ds(0, 16)]
                    v1 = rows_v[buf, b + u, pl.ds(16, 16)]
                    col = jnp.full((16,), 0, jnp.int32) + (b + u)
                    plsc.store_scatter(trans_v, [lanes, col], v0)
                    plsc.store_scatter(trans_v, [lanes + 16, col], v1)
                return c2

            lax.fori_loop(0, RB // UNROLL, b_body, 0)
            pltpu.sync_copy(
                trans_v.at[:, pl.ds(0, RB)], outT_hbm.at[h, :, pl.ds(b0, RB)]
            )

        fire(0, 0, gsem0)

        def pair_body(p, carry):
            h0 = 2 * p
            drain(0, gsem0)
            fire(h0 + 1, 1, gsem1)
            process(h0, 0)
            drain(1, gsem1)

            @pl.when(p < H // 2 - 1)
            def _():
                fire(h0 + 2, 0, gsem0)

            process(h0 + 1, 1)
            return carry

        lax.fori_loop(0, H // 2, pair_body, 0)

    return k(x, data)


def kernel(x, data):
    outT = _sc_gather(x, data)
    return jnp.transpose(outT, (2, 0, 1))


# speedup vs baseline: 1.0085x; 1.0085x over previous
import functools

import jax
import jax.numpy as jnp
from jax import lax
from jax.experimental import pallas as pl
from jax.experimental.pallas import tpu as pltpu
from jax.experimental.pallas import tpu_sc as plsc

B = 16384
H = 50
D = 32
NC = 2
NS = 16
NW = NC * NS
RB = B // NW
TPAD = RB + 8
UNROLL = 8


def _sc_gather(x, data):
    mesh = plsc.VectorSubcoreMesh(core_axis_name="c", subcore_axis_name="s")

    @functools.partial(
        pl.kernel,
        out_type=jax.ShapeDtypeStruct((H, D, B), jnp.float32),
        mesh=mesh,
        scratch_types=[
            pltpu.VMEM((RB * H,), jnp.int32),
            pltpu.VMEM((H, RB), jnp.int32),
            pltpu.VMEM((2, RB, D), jnp.float32),
            pltpu.VMEM((D, TPAD), jnp.float32),
            pltpu.SemaphoreType.DMA,
            pltpu.SemaphoreType.DMA,
        ],
        compiler_params=pltpu.CompilerParams(
            use_tc_tiling_on_sc=False, needs_layout_passes=False
        ),
    )
    def k(x_hbm, data_hbm, outT_hbm, idxBH_v, idxT_v, rows_v, trans_v,
          gsem0, gsem1):
        wid = lax.axis_index("s") * NC + lax.axis_index("c")
        b0 = wid * RB
        pltpu.sync_copy(x_hbm.at[pl.ds(b0 * H, RB * H)], idxBH_v)
        lanes = lax.iota(jnp.int32, 16)

        def t_body(i, c):
            h = i // (RB // 64)
            blk4 = i % (RB // 64)
            for u in range(4):
                bb = (blk4 * 4 + u) * 16
                v = plsc.load_gather(idxBH_v, [(lanes + bb) * H + h])
                idxT_v[h, pl.ds(bb, 16)] = v
            return c

        lax.fori_loop(0, H * (RB // 64), t_body, 0)

        def fire(h, buf, sem):
            pltpu.async_copy(data_hbm.at[idxT_v.at[h]], rows_v.at[buf], sem)

        def drain(buf, sem):
            pltpu.make_async_copy(
                outT_hbm.at[0, :, pl.ds(b0, RB)], rows_v.at[buf], sem
            ).wait()

        def process(h, buf):
            def b_body(bb, c2):
                b = bb * UNROLL
                for u in range(UNROLL):
                    v0 = rows_v[buf, b + u, pl.ds(0, 16)]
                    v1 = rows_v[buf, b + u, pl.ds(16, 16)]
                    col = jnp.full((16,), 0, jnp.int32) + (b + u)
                    plsc.store_scatter(trans_v, [lanes, col], v0)
                    plsc.store_scatter(trans_v, [lanes + 16, col], v1)
                return c2

            lax.fori_loop(0, RB // UNROLL, b_body, 0)
            pltpu.sync_copy(
                trans_v.at[:, pl.ds(0, RB)], outT_hbm.at[h, :, pl.ds(b0, RB)]
            )

        fire(0, 0, gsem0)

        def pair_body(p, carry):
            h0 = 2 * p
            drain(0, gsem0)
            fire(h0 + 1, 1, gsem1)
            process(h0, 0)
            drain(1, gsem1)

            @pl.when(p < H // 2 - 1)
            def _():
                fire(h0 + 2, 0, gsem0)

            process(h0 + 1, 1)
            return carry

        lax.fori_loop(0, H // 2, pair_body, 0)

    return k(x, data)


def kernel(x, data):
    outT = _sc_gather(x.reshape(-1), data)
    return jnp.transpose(outT, (2, 0, 1))


# speedup vs baseline: 1.0511x; 1.0422x over previous
import functools

import jax
import jax.numpy as jnp
from jax import lax
from jax.experimental import pallas as pl
from jax.experimental.pallas import tpu as pltpu
from jax.experimental.pallas import tpu_sc as plsc

B = 16384
H = 50
D = 32
NC = 2
NS = 16
NW = NC * NS
RB = B // NW
TPAD = RB + 8
UNROLL = 8


def _sc_gather(xT, data):
    mesh = plsc.VectorSubcoreMesh(core_axis_name="c", subcore_axis_name="s")

    @functools.partial(
        pl.kernel,
        out_type=jax.ShapeDtypeStruct((H, D, B), jnp.float32),
        mesh=mesh,
        scratch_types=[
            pltpu.VMEM((H, RB), jnp.int32),
            pltpu.VMEM((2, RB, D), jnp.float32),
            pltpu.VMEM((2, D, TPAD), jnp.float32),
            pltpu.SemaphoreType.DMA,
            pltpu.SemaphoreType.DMA,
            pltpu.SemaphoreType.DMA,
            pltpu.SemaphoreType.DMA,
        ],
        compiler_params=pltpu.CompilerParams(
            use_tc_tiling_on_sc=False, needs_layout_passes=False
        ),
    )
    def k(xT_hbm, data_hbm, outT_hbm, idxT_v, rows_v, trans_v,
          gsem0, gsem1, ssem0, ssem1):
        wid = lax.axis_index("s") * NC + lax.axis_index("c")
        b0 = wid * RB
        pltpu.sync_copy(xT_hbm.at[:, pl.ds(b0, RB)], idxT_v)
        lanes = lax.iota(jnp.int32, 16)

        def fire(h, buf, sem):
            pltpu.async_copy(data_hbm.at[idxT_v.at[h]], rows_v.at[buf], sem)

        def drain(buf, sem):
            pltpu.make_async_copy(
                outT_hbm.at[0, :, pl.ds(b0, RB)], rows_v.at[buf], sem
            ).wait()

        def drain_store(tb, sem):
            pltpu.make_async_copy(
                outT_hbm.at[0, :, pl.ds(b0, RB)], rows_v.at[tb], sem
            ).wait()

        def process(h, buf, tb, sem):
            def b_body(bb, c2):
                b = bb * UNROLL
                for u in range(UNROLL):
                    v0 = rows_v[buf, b + u, pl.ds(0, 16)]
                    v1 = rows_v[buf, b + u, pl.ds(16, 16)]
                    col = jnp.full((16,), 0, jnp.int32) + (b + u)
                    plsc.store_scatter(trans_v.at[tb], [lanes, col], v0)
                    plsc.store_scatter(trans_v.at[tb], [lanes + 16, col], v1)
                return c2

            lax.fori_loop(0, RB // UNROLL, b_body, 0)
            pltpu.async_copy(
                trans_v.at[tb, :, pl.ds(0, RB)],
                outT_hbm.at[h, :, pl.ds(b0, RB)], sem,
            )

        fire(0, 0, gsem0)

        def pair_body(p, carry):
            h0 = 2 * p
            drain(0, gsem0)
            fire(h0 + 1, 1, gsem1)

            @pl.when(p > 0)
            def _():
                drain_store(0, ssem0)

            process(h0, 0, 0, ssem0)
            drain(1, gsem1)

            @pl.when(p < H // 2 - 1)
            def _():
                fire(h0 + 2, 0, gsem0)

            @pl.when(p > 0)
            def _():
                drain_store(1, ssem1)

            process(h0 + 1, 1, 1, ssem1)
            return carry

        lax.fori_loop(0, H // 2, pair_body, 0)
        drain_store(0, ssem0)
        drain_store(1, ssem1)

    return k(xT, data)


def kernel(x, data):
    outT = _sc_gather(x.T, data)
    return jnp.transpose(outT, (2, 0, 1))
